# Initial kernel scaffold; baseline (speedup 1.0000x reference)
import functools

import jax
import jax.numpy as jnp
from jax import lax
from jax.experimental import pallas as pl
from jax.experimental.pallas import tpu as pltpu
from jax.experimental.pallas import tpu_sc as plsc

_N = 10000
_E = 320000
_D = 128
_H = 128
_HID = 64
_C = 10
_G = 256

_NP = 10240
_NC = 2
_NS = 16
_NW = _NC * _NS
_RPT = _NP // _NS

_EP = 327680
_KE = 128
_NCHE = (_EP // _NW) // _KE

_KP = 64
_NCHP = (_NP // _NW) // _KP

_CROWS = 512

_RB = 256


def _segment_rows_sc(t, src2, dst2, nch, k):
    mesh = plsc.VectorSubcoreMesh(core_axis_name="c", subcore_axis_name="s")

    @functools.partial(
        pl.kernel,
        out_type=jax.ShapeDtypeStruct((_NC * _NP, _H), jnp.float32),
        mesh=mesh,
        scratch_types=[
            pltpu.VMEM((nch, k), jnp.int32),
            pltpu.VMEM((nch, k), jnp.int32),
            pltpu.VMEM((k, _H), jnp.float32),
            pltpu.VMEM_SHARED((_NP, _H), jnp.float32),
            pltpu.SemaphoreType.DMA,
        ],
    )
    def body(t_hbm, src_hbm, dst_hbm, out_hbm, src_v, dst_v, buf, acc, sem):
        cid = lax.axis_index("c")
        sid = lax.axis_index("s")
        wid = cid * _NS + sid

        @pl.loop(0, k)
        def _(r):
            @pl.loop(0, _H, step=16)
            def _(c0):
                buf[r, pl.ds(c0, 16)] = jnp.zeros((16,), jnp.float32)

        @pl.loop(0, _RPT // k)
        def _(r):
            pltpu.sync_copy(buf, acc.at[pl.ds(sid * _RPT + r * k, k)])

        plsc.subcore_barrier()

        pltpu.async_copy(src_hbm.at[pl.ds(wid * nch, nch)], src_v, sem).wait()
        pltpu.async_copy(dst_hbm.at[pl.ds(wid * nch, nch)], dst_v, sem).wait()

        @pl.loop(0, nch)
        def _(j):
            pltpu.sync_copy(t_hbm.at[src_v.at[j]], buf)
            pltpu.sync_copy(buf, acc.at[dst_v.at[j]], add=True)

        plsc.subcore_barrier()
        pltpu.sync_copy(acc.at[pl.ds(sid * _RPT, _RPT)],
                        out_hbm.at[pl.ds(cid * _NP + sid * _RPT, _RPT)])

    return body(t, src2, dst2)


def _histograms_sc(dst2, bat2):
    mesh = plsc.VectorSubcoreMesh(core_axis_name="c", subcore_axis_name="s")

    @functools.partial(
        pl.kernel,
        out_type=[jax.ShapeDtypeStruct((_NC * _NP, 16), jnp.float32),
                  jax.ShapeDtypeStruct((_NC * _CROWS, 16), jnp.float32)],
        mesh=mesh,
        scratch_types=[
            pltpu.VMEM((_NCHE, _KE), jnp.int32),
            pltpu.VMEM((_NCHP, _KP), jnp.int32),
            pltpu.VMEM((_KE, 16), jnp.float32),
            pltpu.VMEM((_KE, 16), jnp.float32),
            pltpu.VMEM_SHARED((_NP, 16), jnp.float32),
            pltpu.VMEM_SHARED((_CROWS, 16), jnp.float32),
            pltpu.SemaphoreType.DMA,
        ],
    )
    def body(dst_hbm, b_hbm, deg_hbm, cnt_hbm,
             dst_v, b_v, ones_v, z_v, dacc, cacc, sem):
        cid = lax.axis_index("c")
        sid = lax.axis_index("s")
        wid = cid * _NS + sid
        crpt = _CROWS // _NS

        @pl.loop(0, _KE)
        def _(r):
            ones_v[r, pl.ds(0, 16)] = jnp.ones((16,), jnp.float32)
            z_v[r, pl.ds(0, 16)] = jnp.zeros((16,), jnp.float32)

        @pl.loop(0, _RPT // _KE)
        def _(r):
            pltpu.sync_copy(z_v, dacc.at[pl.ds(sid * _RPT + r * _KE, _KE)])
        pltpu.sync_copy(z_v.at[pl.ds(0, crpt)],
                        cacc.at[pl.ds(sid * crpt, crpt)])

        plsc.subcore_barrier()

        pltpu.async_copy(dst_hbm.at[pl.ds(wid * _NCHE, _NCHE)], dst_v, sem).wait()
        pltpu.async_copy(b_hbm.at[pl.ds(wid * _NCHP, _NCHP)], b_v, sem).wait()

        @pl.loop(0, _NCHE)
        def _(j):
            pltpu.sync_copy(ones_v, dacc.at[dst_v.at[j]], add=True)

        @pl.loop(0, _NCHP)
        def _(j):
            pltpu.sync_copy(ones_v.at[pl.ds(0, _KP)], cacc.at[b_v.at[j]], add=True)

        plsc.subcore_barrier()
        pltpu.sync_copy(dacc.at[pl.ds(sid * _RPT, _RPT)],
                        deg_hbm.at[pl.ds(cid * _NP + sid * _RPT, _RPT)])
        pltpu.sync_copy(cacc.at[pl.ds(sid * crpt, crpt)],
                        cnt_hbm.at[pl.ds(cid * _CROWS + sid * crpt, crpt)])

    return body(dst2, bat2)


def _prep_body(deg_ref, x_ref, w_ref, dinv_ref, t0_ref):
    deg = deg_ref[0, :, 0:1] + deg_ref[1, :, 0:1] + 1.0
    db = jnp.broadcast_to(lax.rsqrt(deg), (_RB, _H))
    dinv_ref[...] = db
    t0_ref[...] = db * jnp.dot(x_ref[...], w_ref[...],
                               preferred_element_type=jnp.float32)


def _prep_tc(deg3, x_pad, w0):
    return pl.pallas_call(
        _prep_body,
        grid=(_NP // _RB,),
        in_specs=[pl.BlockSpec((_NC, _RB, 16), lambda i: (0, i, 0)),
                  pl.BlockSpec((_RB, _D), lambda i: (i, 0)),
                  pl.BlockSpec((_D, _H), lambda i: (0, 0))],
        out_specs=[pl.BlockSpec((_RB, _H), lambda i: (i, 0)),
                   pl.BlockSpec((_RB, _H), lambda i: (i, 0))],
        out_shape=[jax.ShapeDtypeStruct((_NP, _H), jnp.float32)] * 2,
    )(deg3, x_pad, w0)


def _combine_mm_body(p_ref, t_ref, dinv_ref, b_ref, w_ref, o_ref):
    s = p_ref[0] + p_ref[1] + t_ref[...]
    h = jnp.maximum(dinv_ref[...] * s + b_ref[0:1, :], 0.0)
    o_ref[...] = dinv_ref[...] * jnp.dot(h, w_ref[...],
                                         preferred_element_type=jnp.float32)


def _combine_mm(p3, t, dinv_b, b8, w):
    return pl.pallas_call(
        _combine_mm_body,
        grid=(_NP // _RB,),
        in_specs=[pl.BlockSpec((_NC, _RB, _H), lambda i: (0, i, 0)),
                  pl.BlockSpec((_RB, _H), lambda i: (i, 0)),
                  pl.BlockSpec((_RB, _H), lambda i: (i, 0)),
                  pl.BlockSpec((8, _H), lambda i: (0, 0)),
                  pl.BlockSpec((_H, _H), lambda i: (0, 0))],
        out_specs=pl.BlockSpec((_RB, _H), lambda i: (i, 0)),
        out_shape=jax.ShapeDtypeStruct((_NP, _H), jnp.float32),
    )(p3, t, dinv_b, b8, w)


def _combine_id_body(p_ref, t_ref, dinv_ref, b_ref, o_ref):
    s = p_ref[0] + p_ref[1] + t_ref[...]
    o_ref[...] = jnp.maximum(dinv_ref[...] * s + b_ref[0:1, :], 0.0)


def _combine_id(p3, t, dinv_b, b8):
    return pl.pallas_call(
        _combine_id_body,
        grid=(_NP // _RB,),
        in_specs=[pl.BlockSpec((_NC, _RB, _H), lambda i: (0, i, 0)),
                  pl.BlockSpec((_RB, _H), lambda i: (i, 0)),
                  pl.BlockSpec((_RB, _H), lambda i: (i, 0)),
                  pl.BlockSpec((8, _H), lambda i: (0, 0))],
        out_specs=pl.BlockSpec((_RB, _H), lambda i: (i, 0)),
        out_shape=jax.ShapeDtypeStruct((_NP, _H), jnp.float32),
    )(p3, t, dinv_b, b8)


def _final_body(p_ref, c_ref, w0_ref, b0_ref, gam_ref, bet_ref, mu_ref,
                var_ref, w1_ref, b1_ref, o_ref):
    pooled = p_ref[0] + p_ref[1]
    cnt = c_ref[0, :, 0:1] + c_ref[1, :, 0:1]
    mean = pooled / jnp.maximum(cnt, 1.0)
    z = jnp.dot(mean, w0_ref[...], preferred_element_type=jnp.float32) + b0_ref[0:1, :]
    z = (z - mu_ref[0:1, :]) * lax.rsqrt(var_ref[0:1, :] + 1e-5) * gam_ref[0:1, :] + bet_ref[0:1, :]
    z = jnp.maximum(z, 0.0)
    out = jnp.dot(z, w1_ref[...], preferred_element_type=jnp.float32) + b1_ref[0:1, :]
    m = jnp.max(out, axis=1, keepdims=True)
    lse = jnp.log(jnp.sum(jnp.exp(out - m), axis=1, keepdims=True)) + m
    o_ref[...] = out - lse


def _final_tc(pp3, cnt3, w0, b0, gam, bet, mu, var, w1, b1):
    return pl.pallas_call(
        _final_body,
        out_shape=jax.ShapeDtypeStruct((_G, _C), jnp.float32),
    )(pp3, cnt3, w0, b0, gam, bet, mu, var, w1, b1)


def kernel(x, edge_index, batch, conv_W0, conv_b0, conv_W1, conv_b1,
           conv_W2, conv_b2, mlp_W0, mlp_b0, bn_gamma, bn_beta, bn_mean,
           bn_var, mlp_W1, mlp_b1):
    src = edge_index[0].astype(jnp.int32)
    dst = edge_index[1].astype(jnp.int32)
    pad_e = _EP - _E
    src_p = jnp.concatenate(
        [src, jnp.zeros((pad_e,), jnp.int32)]).reshape(_NW * _NCHE, _KE)
    dst_p = jnp.concatenate(
        [dst, jnp.full((pad_e,), _NP - 1, jnp.int32)]).reshape(_NW * _NCHE, _KE)

    bat = batch.astype(jnp.int32)
    bat_p = jnp.concatenate(
        [bat, jnp.full((_NP - _N,), _CROWS - 1, jnp.int32)]).reshape(_NW * _NCHP, _KP)
    pool_src = jnp.concatenate(
        [jnp.arange(_N, dtype=jnp.int32),
         jnp.zeros((_NP - _N,), jnp.int32)]).reshape(_NW * _NCHP, _KP)

    x_pad = jnp.pad(x, ((0, _NP - _N), (0, 0)))

    deg_f, cnt_f = _histograms_sc(dst_p, bat_p)
    deg3 = deg_f.reshape(_NC, _NP, 16)
    cnt3 = cnt_f.reshape(_NC, _CROWS, 16)[:, :_G]

    dinv_b, t0 = _prep_tc(deg3, x_pad, conv_W0)

    def b8(v):
        return jnp.tile(v[None, :], (8, 1))

    p0 = _segment_rows_sc(t0, src_p, dst_p, _NCHE, _KE).reshape(_NC, _NP, _H)
    t1 = _combine_mm(p0, t0, dinv_b, b8(conv_b0), conv_W1)
    p1 = _segment_rows_sc(t1, src_p, dst_p, _NCHE, _KE).reshape(_NC, _NP, _H)
    t2 = _combine_mm(p1, t1, dinv_b, b8(conv_b1), conv_W2)
    p2 = _segment_rows_sc(t2, src_p, dst_p, _NCHE, _KE).reshape(_NC, _NP, _H)
    h3 = _combine_id(p2, t2, dinv_b, b8(conv_b2))

    pp3 = _segment_rows_sc(h3, pool_src, bat_p, _NCHP, _KP)
    pp3 = pp3.reshape(_NC, _NP, _H)[:, :_G]

    return _final_tc(pp3, cnt3, mlp_W0, b8(mlp_b0), b8(bn_gamma),
                     b8(bn_beta), b8(bn_mean), b8(bn_var), mlp_W1, b8(mlp_b1))


# speedup vs baseline: 5.8153x; 5.8153x over previous
import functools

import jax
import jax.numpy as jnp
from jax import lax
from jax.experimental import pallas as pl
from jax.experimental.pallas import tpu as pltpu
from jax.experimental.pallas import tpu_sc as plsc

_N = 10000
_E = 320000
_D = 128
_H = 128
_HID = 64
_C = 10
_G = 256

_NP = 10240
_NC = 2
_NS = 16
_NW = _NC * _NS
_RPT = _NP // _NS

_EP = 327680
_KE = 128
_NCHE = (_EP // _NW) // _KE

_KP = 64
_NCHP = 8
_PP = _NW * _NCHP * _KP

_CROWS = 512

_RB = 256


def _segment_rows_sc(t, src2, dst2, zrows, nch, k):
    mesh = plsc.VectorSubcoreMesh(core_axis_name="c", subcore_axis_name="s")

    @functools.partial(
        pl.kernel,
        out_type=jax.ShapeDtypeStruct((_NC * _NP, _H), jnp.float32),
        mesh=mesh,
        scratch_types=[
            pltpu.VMEM((nch, k), jnp.int32),
            pltpu.VMEM((nch, k), jnp.int32),
            pltpu.VMEM((k, _H), jnp.float32),
            pltpu.VMEM_SHARED((_NP, _H), jnp.float32),
            pltpu.SemaphoreType.DMA,
        ],
    )
    def body(t_hbm, src_hbm, dst_hbm, z_hbm, out_hbm, src_v, dst_v, buf, acc, sem):
        cid = lax.axis_index("c")
        sid = lax.axis_index("s")
        wid = cid * _NS + sid

        pltpu.sync_copy(z_hbm, acc.at[pl.ds(sid * _RPT, _RPT)])

        plsc.subcore_barrier()

        pltpu.async_copy(src_hbm.at[pl.ds(wid * nch, nch)], src_v, sem).wait()
        pltpu.async_copy(dst_hbm.at[pl.ds(wid * nch, nch)], dst_v, sem).wait()

        @pl.loop(0, nch)
        def _(j):
            pltpu.sync_copy(t_hbm.at[src_v.at[j]], buf)
            pltpu.sync_copy(buf, acc.at[dst_v.at[j]], add=True)

        plsc.subcore_barrier()
        pltpu.sync_copy(acc.at[pl.ds(sid * _RPT, _RPT)],
                        out_hbm.at[pl.ds(cid * _NP + sid * _RPT, _RPT)])

    return body(t, src2, dst2, zrows)


def _histograms_sc(dst2, bat2, z16, o16):
    mesh = plsc.VectorSubcoreMesh(core_axis_name="c", subcore_axis_name="s")

    @functools.partial(
        pl.kernel,
        out_type=[jax.ShapeDtypeStruct((_NC * _NP, _H), jnp.float32),
                  jax.ShapeDtypeStruct((_NC * _CROWS, _H), jnp.float32)],
        mesh=mesh,
        scratch_types=[
            pltpu.VMEM((_NCHE, _KE), jnp.int32),
            pltpu.VMEM((_NCHP, _KP), jnp.int32),
            pltpu.VMEM((_KE, _H), jnp.float32),
            pltpu.VMEM_SHARED((_NP, _H), jnp.float32),
            pltpu.VMEM_SHARED((_CROWS, _H), jnp.float32),
            pltpu.SemaphoreType.DMA,
        ],
    )
    def body(dst_hbm, b_hbm, z_hbm, o_hbm, deg_hbm, cnt_hbm,
             dst_v, b_v, ones_v, dacc, cacc, sem):
        cid = lax.axis_index("c")
        sid = lax.axis_index("s")
        wid = cid * _NS + sid
        crpt = _CROWS // _NS

        pltpu.sync_copy(o_hbm, ones_v)
        pltpu.sync_copy(z_hbm, dacc.at[pl.ds(sid * _RPT, _RPT)])
        pltpu.sync_copy(z_hbm.at[pl.ds(0, crpt)],
                        cacc.at[pl.ds(sid * crpt, crpt)])

        plsc.subcore_barrier()

        pltpu.async_copy(dst_hbm.at[pl.ds(wid * _NCHE, _NCHE)], dst_v, sem).wait()
        pltpu.async_copy(b_hbm.at[pl.ds(wid * _NCHP, _NCHP)], b_v, sem).wait()

        @pl.loop(0, _NCHE)
        def _(j):
            pltpu.sync_copy(ones_v, dacc.at[dst_v.at[j]], add=True)

        @pl.loop(0, _NCHP)
        def _(j):
            pltpu.sync_copy(ones_v.at[pl.ds(0, _KP)], cacc.at[b_v.at[j]], add=True)

        plsc.subcore_barrier()
        pltpu.sync_copy(dacc.at[pl.ds(sid * _RPT, _RPT)],
                        deg_hbm.at[pl.ds(cid * _NP + sid * _RPT, _RPT)])
        pltpu.sync_copy(cacc.at[pl.ds(sid * crpt, crpt)],
                        cnt_hbm.at[pl.ds(cid * _CROWS + sid * crpt, crpt)])

    return body(dst2, bat2, z16, o16)


def _prep_body(deg_ref, x_ref, w_ref, dinv_ref, t0_ref):
    deg = deg_ref[0, :, 0:1] + deg_ref[1, :, 0:1] + 1.0
    db = jnp.broadcast_to(lax.rsqrt(deg), (_RB, _H))
    dinv_ref[...] = db
    t0_ref[...] = db * jnp.dot(x_ref[...], w_ref[...],
                               preferred_element_type=jnp.float32)


def _prep_tc(deg3, x_pad, w0):
    return pl.pallas_call(
        _prep_body,
        grid=(_NP // _RB,),
        in_specs=[pl.BlockSpec((_NC, _RB, _H), lambda i: (0, i, 0)),
                  pl.BlockSpec((_RB, _D), lambda i: (i, 0)),
                  pl.BlockSpec((_D, _H), lambda i: (0, 0))],
        out_specs=[pl.BlockSpec((_RB, _H), lambda i: (i, 0)),
                   pl.BlockSpec((_RB, _H), lambda i: (i, 0))],
        out_shape=[jax.ShapeDtypeStruct((_NP, _H), jnp.float32)] * 2,
    )(deg3, x_pad, w0)


def _combine_mm_body(p_ref, t_ref, dinv_ref, b_ref, w_ref, o_ref):
    s = p_ref[0] + p_ref[1] + t_ref[...]
    h = jnp.maximum(dinv_ref[...] * s + b_ref[0:1, :], 0.0)
    o_ref[...] = dinv_ref[...] * jnp.dot(h, w_ref[...],
                                         preferred_element_type=jnp.float32)


def _combine_mm(p3, t, dinv_b, b8, w):
    return pl.pallas_call(
        _combine_mm_body,
        grid=(_NP // _RB,),
        in_specs=[pl.BlockSpec((_NC, _RB, _H), lambda i: (0, i, 0)),
                  pl.BlockSpec((_RB, _H), lambda i: (i, 0)),
                  pl.BlockSpec((_RB, _H), lambda i: (i, 0)),
                  pl.BlockSpec((8, _H), lambda i: (0, 0)),
                  pl.BlockSpec((_H, _H), lambda i: (0, 0))],
        out_specs=pl.BlockSpec((_RB, _H), lambda i: (i, 0)),
        out_shape=jax.ShapeDtypeStruct((_NP, _H), jnp.float32),
    )(p3, t, dinv_b, b8, w)


def _combine_id_body(p_ref, t_ref, dinv_ref, b_ref, o_ref):
    s = p_ref[0] + p_ref[1] + t_ref[...]
    o_ref[...] = jnp.maximum(dinv_ref[...] * s + b_ref[0:1, :], 0.0)


def _combine_id(p3, t, dinv_b, b8):
    return pl.pallas_call(
        _combine_id_body,
        grid=(_NP // _RB,),
        in_specs=[pl.BlockSpec((_NC, _RB, _H), lambda i: (0, i, 0)),
                  pl.BlockSpec((_RB, _H), lambda i: (i, 0)),
                  pl.BlockSpec((_RB, _H), lambda i: (i, 0)),
                  pl.BlockSpec((8, _H), lambda i: (0, 0))],
        out_specs=pl.BlockSpec((_RB, _H), lambda i: (i, 0)),
        out_shape=jax.ShapeDtypeStruct((_NP, _H), jnp.float32),
    )(p3, t, dinv_b, b8)


def _final_body(p_ref, c_ref, w0_ref, b0_ref, gam_ref, bet_ref, mu_ref,
                var_ref, w1_ref, b1_ref, o_ref):
    pooled = p_ref[0] + p_ref[1]
    cnt = c_ref[0, :, 0:1] + c_ref[1, :, 0:1]
    mean = pooled / jnp.maximum(cnt, 1.0)
    z = jnp.dot(mean, w0_ref[...], preferred_element_type=jnp.float32) + b0_ref[0:1, :]
    z = (z - mu_ref[0:1, :]) * lax.rsqrt(var_ref[0:1, :] + 1e-5) * gam_ref[0:1, :] + bet_ref[0:1, :]
    z = jnp.maximum(z, 0.0)
    out = jnp.dot(z, w1_ref[...], preferred_element_type=jnp.float32) + b1_ref[0:1, :]
    m = jnp.max(out, axis=1, keepdims=True)
    lse = jnp.log(jnp.sum(jnp.exp(out - m), axis=1, keepdims=True)) + m
    o_ref[...] = out - lse


def _final_tc(pp3, cnt3, w0, b0, gam, bet, mu, var, w1, b1):
    return pl.pallas_call(
        _final_body,
        out_shape=jax.ShapeDtypeStruct((_G, _C), jnp.float32),
    )(pp3, cnt3, w0, b0, gam, bet, mu, var, w1, b1)


def kernel(x, edge_index, batch, conv_W0, conv_b0, conv_W1, conv_b1,
           conv_W2, conv_b2, mlp_W0, mlp_b0, bn_gamma, bn_beta, bn_mean,
           bn_var, mlp_W1, mlp_b1):
    src = edge_index[0].astype(jnp.int32)
    dst = edge_index[1].astype(jnp.int32)
    pad_e = _EP - _E
    src_p = jnp.concatenate(
        [src, jnp.zeros((pad_e,), jnp.int32)]).reshape(_NW * _NCHE, _KE)
    dst_p = jnp.concatenate(
        [dst, jnp.full((pad_e,), _NP - 1, jnp.int32)]).reshape(_NW * _NCHE, _KE)

    bat = batch.astype(jnp.int32)
    bat_p = jnp.concatenate(
        [bat, jnp.full((_PP - _N,), _CROWS - 1, jnp.int32)]).reshape(_NW * _NCHP, _KP)
    pool_src = jnp.concatenate(
        [jnp.arange(_N, dtype=jnp.int32),
         jnp.zeros((_PP - _N,), jnp.int32)]).reshape(_NW * _NCHP, _KP)

    x_pad = jnp.pad(x, ((0, _NP - _N), (0, 0)))

    zrows = jnp.zeros((_RPT, _H), jnp.float32)
    o128 = jnp.ones((_KE, _H), jnp.float32)

    deg_f, cnt_f = _histograms_sc(dst_p, bat_p, zrows, o128)
    deg3 = deg_f.reshape(_NC, _NP, _H)
    cnt3 = cnt_f.reshape(_NC, _CROWS, _H)[:, :_G]

    dinv_b, t0 = _prep_tc(deg3, x_pad, conv_W0)

    def b8(v):
        return jnp.tile(v[None, :], (8, 1))

    p0 = _segment_rows_sc(t0, src_p, dst_p, zrows, _NCHE, _KE).reshape(_NC, _NP, _H)
    t1 = _combine_mm(p0, t0, dinv_b, b8(conv_b0), conv_W1)
    p1 = _segment_rows_sc(t1, src_p, dst_p, zrows, _NCHE, _KE).reshape(_NC, _NP, _H)
    t2 = _combine_mm(p1, t1, dinv_b, b8(conv_b1), conv_W2)
    p2 = _segment_rows_sc(t2, src_p, dst_p, zrows, _NCHE, _KE).reshape(_NC, _NP, _H)
    h3 = _combine_id(p2, t2, dinv_b, b8(conv_b2))

    pp3 = _segment_rows_sc(h3, pool_src, bat_p, zrows, _NCHP, _KP)
    pp3 = pp3.reshape(_NC, _NP, _H)[:, :_G]

    return _final_tc(pp3, cnt3, mlp_W0, b8(mlp_b0), b8(bn_gamma),
                     b8(bn_beta), b8(bn_mean), b8(bn_var), mlp_W1, b8(mlp_b1))


# speedup vs baseline: 6.1232x; 1.0529x over previous
import functools

import jax
import jax.numpy as jnp
from jax import lax
from jax.experimental import pallas as pl
from jax.experimental.pallas import tpu as pltpu
from jax.experimental.pallas import tpu_sc as plsc

_N = 10000
_E = 320000
_D = 128
_H = 128
_HID = 64
_C = 10
_G = 256

_NP = 10240
_NC = 2
_NS = 16
_NW = _NC * _NS
_RPT = _NP // _NS

_EP = 327680
_KE = 128
_NCHE = (_EP // _NW) // _KE

_KP = 64
_NCHP = 8
_PP = _NW * _NCHP * _KP

_CROWS = 512

_RB = 256


def _segment_rows_sc(t, src2, dst2, zrows, nch, k):
    mesh = plsc.VectorSubcoreMesh(core_axis_name="c", subcore_axis_name="s")
    ib = min(16, nch)
    nblk = nch // ib

    @functools.partial(
        pl.kernel,
        out_type=jax.ShapeDtypeStruct((_NC * _NP, _H), jnp.float32),
        mesh=mesh,
        scratch_types=[
            pltpu.VMEM((ib, k), jnp.int32),
            pltpu.VMEM((ib, k), jnp.int32),
            pltpu.VMEM((k, _H), jnp.float32),
            pltpu.VMEM((k, _H), jnp.float32),
            pltpu.VMEM_SHARED((_NP, _H), jnp.float32),
            pltpu.SemaphoreType.DMA,
            pltpu.SemaphoreType.DMA,
        ],
    )
    def body(t_hbm, src_hbm, dst_hbm, z_hbm, out_hbm, src_v, dst_v, buf_a,
             buf_b, acc, sem_a, sem_b):
        cid = lax.axis_index("c")
        sid = lax.axis_index("s")
        wid = cid * _NS + sid

        pltpu.sync_copy(z_hbm, acc.at[pl.ds(sid * _RPT, _RPT)])

        plsc.subcore_barrier()

        @pl.loop(0, nblk)
        def _(b):
            base = wid * nch + b * ib
            pltpu.async_copy(src_hbm.at[pl.ds(base, ib)], src_v, sem_a).wait()
            pltpu.async_copy(dst_hbm.at[pl.ds(base, ib)], dst_v, sem_a).wait()
            pltpu.async_copy(t_hbm.at[src_v.at[0]], buf_a, sem_a)

            @pl.loop(0, ib // 2)
            def _(jh):
                j = jh * 2
                pltpu.make_async_copy(t_hbm.at[src_v.at[0]], buf_a, sem_a).wait()
                pltpu.async_copy(t_hbm.at[src_v.at[j + 1]], buf_b, sem_b)
                pltpu.sync_copy(buf_a, acc.at[dst_v.at[j]], add=True)
                pltpu.make_async_copy(t_hbm.at[src_v.at[0]], buf_b, sem_b).wait()

                @pl.when(j + 2 < ib)
                def _():
                    pltpu.async_copy(t_hbm.at[src_v.at[j + 2]], buf_a, sem_a)

                pltpu.sync_copy(buf_b, acc.at[dst_v.at[j + 1]], add=True)

        plsc.subcore_barrier()
        pltpu.sync_copy(acc.at[pl.ds(sid * _RPT, _RPT)],
                        out_hbm.at[pl.ds(cid * _NP + sid * _RPT, _RPT)])

    return body(t, src2, dst2, zrows)


def _histograms_sc(dst2, bat2, z16, o16):
    mesh = plsc.VectorSubcoreMesh(core_axis_name="c", subcore_axis_name="s")

    @functools.partial(
        pl.kernel,
        out_type=[jax.ShapeDtypeStruct((_NC * _NP, _H), jnp.float32),
                  jax.ShapeDtypeStruct((_NC * _CROWS, _H), jnp.float32)],
        mesh=mesh,
        scratch_types=[
            pltpu.VMEM((_NCHE, _KE), jnp.int32),
            pltpu.VMEM((_NCHP, _KP), jnp.int32),
            pltpu.VMEM((_KE, _H), jnp.float32),
            pltpu.VMEM_SHARED((_NP, _H), jnp.float32),
            pltpu.VMEM_SHARED((_CROWS, _H), jnp.float32),
            pltpu.SemaphoreType.DMA,
        ],
    )
    def body(dst_hbm, b_hbm, z_hbm, o_hbm, deg_hbm, cnt_hbm,
             dst_v, b_v, ones_v, dacc, cacc, sem):
        cid = lax.axis_index("c")
        sid = lax.axis_index("s")
        wid = cid * _NS + sid
        crpt = _CROWS // _NS

        pltpu.sync_copy(o_hbm, ones_v)
        pltpu.sync_copy(z_hbm, dacc.at[pl.ds(sid * _RPT, _RPT)])
        pltpu.sync_copy(z_hbm.at[pl.ds(0, crpt)],
                        cacc.at[pl.ds(sid * crpt, crpt)])

        plsc.subcore_barrier()

        pltpu.async_copy(dst_hbm.at[pl.ds(wid * _NCHE, _NCHE)], dst_v, sem).wait()
        pltpu.async_copy(b_hbm.at[pl.ds(wid * _NCHP, _NCHP)], b_v, sem).wait()

        @pl.loop(0, _NCHE)
        def _(j):
            pltpu.sync_copy(ones_v, dacc.at[dst_v.at[j]], add=True)

        @pl.loop(0, _NCHP)
        def _(j):
            pltpu.sync_copy(ones_v.at[pl.ds(0, _KP)], cacc.at[b_v.at[j]], add=True)

        plsc.subcore_barrier()
        pltpu.sync_copy(dacc.at[pl.ds(sid * _RPT, _RPT)],
                        deg_hbm.at[pl.ds(cid * _NP + sid * _RPT, _RPT)])
        pltpu.sync_copy(cacc.at[pl.ds(sid * crpt, crpt)],
                        cnt_hbm.at[pl.ds(cid * _CROWS + sid * crpt, crpt)])

    return body(dst2, bat2, z16, o16)


def _prep_body(deg_ref, x_ref, w_ref, dinv_ref, t0_ref):
    deg = deg_ref[0, :, 0:1] + deg_ref[1, :, 0:1] + 1.0
    db = jnp.broadcast_to(lax.rsqrt(deg), (_RB, _H))
    dinv_ref[...] = db
    t0_ref[...] = db * jnp.dot(x_ref[...], w_ref[...],
                               preferred_element_type=jnp.float32)


def _prep_tc(deg3, x_pad, w0):
    return pl.pallas_call(
        _prep_body,
        grid=(_NP // _RB,),
        in_specs=[pl.BlockSpec((_NC, _RB, _H), lambda i: (0, i, 0)),
                  pl.BlockSpec((_RB, _D), lambda i: (i, 0)),
                  pl.BlockSpec((_D, _H), lambda i: (0, 0))],
        out_specs=[pl.BlockSpec((_RB, _H), lambda i: (i, 0)),
                   pl.BlockSpec((_RB, _H), lambda i: (i, 0))],
        out_shape=[jax.ShapeDtypeStruct((_NP, _H), jnp.float32)] * 2,
    )(deg3, x_pad, w0)


def _combine_mm_body(p_ref, t_ref, dinv_ref, b_ref, w_ref, o_ref):
    s = p_ref[0] + p_ref[1] + t_ref[...]
    h = jnp.maximum(dinv_ref[...] * s + b_ref[0:1, :], 0.0)
    o_ref[...] = dinv_ref[...] * jnp.dot(h, w_ref[...],
                                         preferred_element_type=jnp.float32)


def _combine_mm(p3, t, dinv_b, b8, w):
    return pl.pallas_call(
        _combine_mm_body,
        grid=(_NP // _RB,),
        in_specs=[pl.BlockSpec((_NC, _RB, _H), lambda i: (0, i, 0)),
                  pl.BlockSpec((_RB, _H), lambda i: (i, 0)),
                  pl.BlockSpec((_RB, _H), lambda i: (i, 0)),
                  pl.BlockSpec((8, _H), lambda i: (0, 0)),
                  pl.BlockSpec((_H, _H), lambda i: (0, 0))],
        out_specs=pl.BlockSpec((_RB, _H), lambda i: (i, 0)),
        out_shape=jax.ShapeDtypeStruct((_NP, _H), jnp.float32),
    )(p3, t, dinv_b, b8, w)


def _combine_id_body(p_ref, t_ref, dinv_ref, b_ref, o_ref):
    s = p_ref[0] + p_ref[1] + t_ref[...]
    o_ref[...] = jnp.maximum(dinv_ref[...] * s + b_ref[0:1, :], 0.0)


def _combine_id(p3, t, dinv_b, b8):
    return pl.pallas_call(
        _combine_id_body,
        grid=(_NP // _RB,),
        in_specs=[pl.BlockSpec((_NC, _RB, _H), lambda i: (0, i, 0)),
                  pl.BlockSpec((_RB, _H), lambda i: (i, 0)),
                  pl.BlockSpec((_RB, _H), lambda i: (i, 0)),
                  pl.BlockSpec((8, _H), lambda i: (0, 0))],
        out_specs=pl.BlockSpec((_RB, _H), lambda i: (i, 0)),
        out_shape=jax.ShapeDtypeStruct((_NP, _H), jnp.float32),
    )(p3, t, dinv_b, b8)


def _final_body(p_ref, c_ref, w0_ref, b0_ref, gam_ref, bet_ref, mu_ref,
                var_ref, w1_ref, b1_ref, o_ref):
    pooled = p_ref[0] + p_ref[1]
    cnt = c_ref[0, :, 0:1] + c_ref[1, :, 0:1]
    mean = pooled / jnp.maximum(cnt, 1.0)
    z = jnp.dot(mean, w0_ref[...], preferred_element_type=jnp.float32) + b0_ref[0:1, :]
    z = (z - mu_ref[0:1, :]) * lax.rsqrt(var_ref[0:1, :] + 1e-5) * gam_ref[0:1, :] + bet_ref[0:1, :]
    z = jnp.maximum(z, 0.0)
    out = jnp.dot(z, w1_ref[...], preferred_element_type=jnp.float32) + b1_ref[0:1, :]
    m = jnp.max(out, axis=1, keepdims=True)
    lse = jnp.log(jnp.sum(jnp.exp(out - m), axis=1, keepdims=True)) + m
    o_ref[...] = out - lse


def _final_tc(pp3, cnt3, w0, b0, gam, bet, mu, var, w1, b1):
    return pl.pallas_call(
        _final_body,
        out_shape=jax.ShapeDtypeStruct((_G, _C), jnp.float32),
    )(pp3, cnt3, w0, b0, gam, bet, mu, var, w1, b1)


def kernel(x, edge_index, batch, conv_W0, conv_b0, conv_W1, conv_b1,
           conv_W2, conv_b2, mlp_W0, mlp_b0, bn_gamma, bn_beta, bn_mean,
           bn_var, mlp_W1, mlp_b1):
    src = edge_index[0].astype(jnp.int32)
    dst = edge_index[1].astype(jnp.int32)
    pad_e = _EP - _E
    src_p = jnp.concatenate(
        [src, jnp.zeros((pad_e,), jnp.int32)]).reshape(_NW * _NCHE, _KE)
    dst_p = jnp.concatenate(
        [dst, jnp.full((pad_e,), _NP - 1, jnp.int32)]).reshape(_NW * _NCHE, _KE)

    bat = batch.astype(jnp.int32)
    bat_p = jnp.concatenate(
        [bat, jnp.full((_PP - _N,), _CROWS - 1, jnp.int32)]).reshape(_NW * _NCHP, _KP)
    pool_src = jnp.concatenate(
        [jnp.arange(_N, dtype=jnp.int32),
         jnp.zeros((_PP - _N,), jnp.int32)]).reshape(_NW * _NCHP, _KP)

    x_pad = jnp.pad(x, ((0, _NP - _N), (0, 0)))

    zrows = jnp.zeros((_RPT, _H), jnp.float32)
    o128 = jnp.ones((_KE, _H), jnp.float32)

    deg_f, cnt_f = _histograms_sc(dst_p, bat_p, zrows, o128)
    deg3 = deg_f.reshape(_NC, _NP, _H)
    cnt3 = cnt_f.reshape(_NC, _CROWS, _H)[:, :_G]

    dinv_b, t0 = _prep_tc(deg3, x_pad, conv_W0)

    def b8(v):
        return jnp.tile(v[None, :], (8, 1))

    p0 = _segment_rows_sc(t0, src_p, dst_p, zrows, _NCHE, _KE).reshape(_NC, _NP, _H)
    t1 = _combine_mm(p0, t0, dinv_b, b8(conv_b0), conv_W1)
    p1 = _segment_rows_sc(t1, src_p, dst_p, zrows, _NCHE, _KE).reshape(_NC, _NP, _H)
    t2 = _combine_mm(p1, t1, dinv_b, b8(conv_b1), conv_W2)
    p2 = _segment_rows_sc(t2, src_p, dst_p, zrows, _NCHE, _KE).reshape(_NC, _NP, _H)
    h3 = _combine_id(p2, t2, dinv_b, b8(conv_b2))

    pp3 = _segment_rows_sc(h3, pool_src, bat_p, zrows, _NCHP, _KP)
    pp3 = pp3.reshape(_NC, _NP, _H)[:, :_G]

    return _final_tc(pp3, cnt3, mlp_W0, b8(mlp_b0), b8(bn_gamma),
                     b8(bn_beta), b8(bn_mean), b8(bn_var), mlp_W1, b8(mlp_b1))


# speedup vs baseline: 6.1327x; 1.0015x over previous
import functools

import jax
import jax.numpy as jnp
from jax import lax
from jax.experimental import pallas as pl
from jax.experimental.pallas import tpu as pltpu
from jax.experimental.pallas import tpu_sc as plsc

_N = 10000
_E = 320000
_D = 128
_H = 128
_HID = 64
_C = 10
_G = 256

_NP = 10240
_NC = 2
_NS = 16
_NW = _NC * _NS
_RPT = _NP // _NS

_EP = 327680
_KE = 128
_NCHE = (_EP // _NW) // _KE

_KP = 64
_NCHP = 8
_PP = _NW * _NCHP * _KP

_CROWS = 512

_RB = 256


def _segment_rows_sc(t, src2, dst2, zrows, nch, k):
    mesh = plsc.VectorSubcoreMesh(core_axis_name="c", subcore_axis_name="s")
    ib = min(16, nch)
    nblk = nch // ib

    @functools.partial(
        pl.kernel,
        out_type=jax.ShapeDtypeStruct((_NC * _NP, _H), jnp.float32),
        mesh=mesh,
        scratch_types=[
            pltpu.VMEM((ib, k), jnp.int32),
            pltpu.VMEM((ib, k), jnp.int32),
            pltpu.VMEM((k, _H), jnp.float32),
            pltpu.VMEM((k, _H), jnp.float32),
            pltpu.VMEM_SHARED((_NP, _H), jnp.float32),
            pltpu.SemaphoreType.DMA,
            pltpu.SemaphoreType.DMA,
        ],
    )
    def body(t_hbm, src_hbm, dst_hbm, z_hbm, out_hbm, src_v, dst_v, buf_a,
             buf_b, acc, sem_a, sem_b):
        cid = lax.axis_index("c")
        sid = lax.axis_index("s")
        wid = cid * _NS + sid

        pltpu.sync_copy(z_hbm, acc.at[pl.ds(sid * _RPT, _RPT)])

        plsc.subcore_barrier()

        @pl.loop(0, nblk)
        def _(b):
            base = wid * nch + b * ib
            pltpu.async_copy(src_hbm.at[pl.ds(base, ib)], src_v, sem_a).wait()
            pltpu.async_copy(dst_hbm.at[pl.ds(base, ib)], dst_v, sem_a).wait()
            pltpu.async_copy(t_hbm.at[src_v.at[0]], buf_a, sem_a)

            @pl.loop(0, ib // 2)
            def _(jh):
                j = jh * 2
                pltpu.make_async_copy(t_hbm.at[src_v.at[0]], buf_a, sem_a).wait()
                pltpu.async_copy(t_hbm.at[src_v.at[j + 1]], buf_b, sem_b)
                pltpu.sync_copy(buf_a, acc.at[dst_v.at[j]], add=True)
                pltpu.make_async_copy(t_hbm.at[src_v.at[0]], buf_b, sem_b).wait()

                @pl.when(j + 2 < ib)
                def _():
                    pltpu.async_copy(t_hbm.at[src_v.at[j + 2]], buf_a, sem_a)

                pltpu.sync_copy(buf_b, acc.at[dst_v.at[j + 1]], add=True)

        plsc.subcore_barrier()
        pltpu.sync_copy(acc.at[pl.ds(sid * _RPT, _RPT)],
                        out_hbm.at[pl.ds(cid * _NP + sid * _RPT, _RPT)])

    return body(t, src2, dst2, zrows)


def _histograms_sc(dst2, bat2, z16, o16):
    mesh = plsc.VectorSubcoreMesh(core_axis_name="c", subcore_axis_name="s")

    @functools.partial(
        pl.kernel,
        out_type=[jax.ShapeDtypeStruct((_NC * _NP, _H), jnp.float32),
                  jax.ShapeDtypeStruct((_NC * _CROWS, _H), jnp.float32)],
        mesh=mesh,
        scratch_types=[
            pltpu.VMEM((_NCHE, _KE), jnp.int32),
            pltpu.VMEM((_NCHP, _KP), jnp.int32),
            pltpu.VMEM((_KE, _H), jnp.float32),
            pltpu.VMEM_SHARED((_NP, _H), jnp.float32),
            pltpu.VMEM_SHARED((_CROWS, _H), jnp.float32),
            pltpu.SemaphoreType.DMA,
        ],
    )
    def body(dst_hbm, b_hbm, z_hbm, o_hbm, deg_hbm, cnt_hbm,
             dst_v, b_v, ones_v, dacc, cacc, sem):
        cid = lax.axis_index("c")
        sid = lax.axis_index("s")
        wid = cid * _NS + sid
        crpt = _CROWS // _NS

        pltpu.sync_copy(o_hbm, ones_v)
        pltpu.sync_copy(z_hbm, dacc.at[pl.ds(sid * _RPT, _RPT)])
        pltpu.sync_copy(z_hbm.at[pl.ds(0, crpt)],
                        cacc.at[pl.ds(sid * crpt, crpt)])

        plsc.subcore_barrier()

        pltpu.async_copy(dst_hbm.at[pl.ds(wid * _NCHE, _NCHE)], dst_v, sem).wait()
        pltpu.async_copy(b_hbm.at[pl.ds(wid * _NCHP, _NCHP)], b_v, sem).wait()

        @pl.loop(0, _NCHE)
        def _(j):
            pltpu.sync_copy(ones_v, dacc.at[dst_v.at[j]], add=True)

        @pl.loop(0, _NCHP)
        def _(j):
            pltpu.sync_copy(ones_v.at[pl.ds(0, _KP)], cacc.at[b_v.at[j]], add=True)

        plsc.subcore_barrier()
        pltpu.sync_copy(dacc.at[pl.ds(sid * _RPT, _RPT)],
                        deg_hbm.at[pl.ds(cid * _NP + sid * _RPT, _RPT)])
        pltpu.sync_copy(cacc.at[pl.ds(sid * crpt, crpt)],
                        cnt_hbm.at[pl.ds(cid * _CROWS + sid * crpt, crpt)])

    return body(dst2, bat2, z16, o16)


def _prep_body(deg_ref, x_ref, w_ref, dinv_ref, t0_ref):
    deg = deg_ref[0, :, 0:1] + deg_ref[1, :, 0:1] + 1.0
    db = jnp.broadcast_to(lax.rsqrt(deg), (_RB, _H))
    dinv_ref[...] = db
    t0_ref[...] = db * jnp.dot(x_ref[...], w_ref[...],
                               preferred_element_type=jnp.float32)


def _prep_tc(deg3, x_pad, w0):
    return pl.pallas_call(
        _prep_body,
        grid=(_NP // _RB,),
        in_specs=[pl.BlockSpec((_NC, _RB, _H), lambda i: (0, i, 0)),
                  pl.BlockSpec((_RB, _D), lambda i: (i, 0)),
                  pl.BlockSpec((_D, _H), lambda i: (0, 0))],
        out_specs=[pl.BlockSpec((_RB, _H), lambda i: (i, 0)),
                   pl.BlockSpec((_RB, _H), lambda i: (i, 0))],
        out_shape=[jax.ShapeDtypeStruct((_NP, _H), jnp.float32)] * 2,
    )(deg3, x_pad, w0)


def _combine_mm_body(p_ref, t_ref, dinv_ref, b_ref, w_ref, o_ref):
    s = p_ref[0] + p_ref[1] + t_ref[...]
    h = jnp.maximum(dinv_ref[...] * s + b_ref[0:1, :], 0.0)
    o_ref[...] = dinv_ref[...] * jnp.dot(h, w_ref[...],
                                         preferred_element_type=jnp.float32)


def _combine_mm(p3, t, dinv_b, b8, w):
    return pl.pallas_call(
        _combine_mm_body,
        grid=(_NP // _RB,),
        in_specs=[pl.BlockSpec((_NC, _RB, _H), lambda i: (0, i, 0)),
                  pl.BlockSpec((_RB, _H), lambda i: (i, 0)),
                  pl.BlockSpec((_RB, _H), lambda i: (i, 0)),
                  pl.BlockSpec((8, _H), lambda i: (0, 0)),
                  pl.BlockSpec((_H, _H), lambda i: (0, 0))],
        out_specs=pl.BlockSpec((_RB, _H), lambda i: (i, 0)),
        out_shape=jax.ShapeDtypeStruct((_NP, _H), jnp.float32),
    )(p3, t, dinv_b, b8, w)


def _combine_id_body(p_ref, t_ref, dinv_ref, b_ref, o_ref):
    s = p_ref[0] + p_ref[1] + t_ref[...]
    o_ref[...] = jnp.maximum(dinv_ref[...] * s + b_ref[0:1, :], 0.0)


def _combine_id(p3, t, dinv_b, b8):
    return pl.pallas_call(
        _combine_id_body,
        grid=(_NP // _RB,),
        in_specs=[pl.BlockSpec((_NC, _RB, _H), lambda i: (0, i, 0)),
                  pl.BlockSpec((_RB, _H), lambda i: (i, 0)),
                  pl.BlockSpec((_RB, _H), lambda i: (i, 0)),
                  pl.BlockSpec((8, _H), lambda i: (0, 0))],
        out_specs=pl.BlockSpec((_RB, _H), lambda i: (i, 0)),
        out_shape=jax.ShapeDtypeStruct((_NP, _H), jnp.float32),
    )(p3, t, dinv_b, b8)


def _final_body(p_ref, c_ref, w0_ref, b0_ref, gam_ref, bet_ref, mu_ref,
                var_ref, w1_ref, b1_ref, o_ref):
    pooled = p_ref[0] + p_ref[1]
    cnt = c_ref[0, :, 0:1] + c_ref[1, :, 0:1]
    mean = pooled / jnp.maximum(cnt, 1.0)
    z = jnp.dot(mean, w0_ref[...], preferred_element_type=jnp.float32) + b0_ref[0:1, :]
    z = (z - mu_ref[0:1, :]) * lax.rsqrt(var_ref[0:1, :] + 1e-5) * gam_ref[0:1, :] + bet_ref[0:1, :]
    z = jnp.maximum(z, 0.0)
    out = jnp.dot(z, w1_ref[...], preferred_element_type=jnp.float32) + b1_ref[0:1, :]
    m = jnp.max(out, axis=1, keepdims=True)
    lse = jnp.log(jnp.sum(jnp.exp(out - m), axis=1, keepdims=True)) + m
    o_ref[...] = out - lse


def _final_tc(pp3, cnt3, w0, b0, gam, bet, mu, var, w1, b1):
    return pl.pallas_call(
        _final_body,
        out_shape=jax.ShapeDtypeStruct((_G, _C), jnp.float32),
    )(pp3, cnt3, w0, b0, gam, bet, mu, var, w1, b1)


def kernel(x, edge_index, batch, conv_W0, conv_b0, conv_W1, conv_b1,
           conv_W2, conv_b2, mlp_W0, mlp_b0, bn_gamma, bn_beta, bn_mean,
           bn_var, mlp_W1, mlp_b1):
    src = edge_index[0].astype(jnp.int32)
    dst = edge_index[1].astype(jnp.int32)
    pad_e = _EP - _E
    src_p = jnp.concatenate(
        [src, jnp.zeros((pad_e,), jnp.int32)]).reshape(_NW * _NCHE, _KE)
    pad_dst = _N + jnp.arange(pad_e, dtype=jnp.int32) % (_NP - _N)
    dst_p = jnp.concatenate([dst, pad_dst]).reshape(_NW * _NCHE, _KE)

    bat = batch.astype(jnp.int32)
    pad_bat = _G + jnp.arange(_PP - _N, dtype=jnp.int32) % (_CROWS - _G)
    bat_p = jnp.concatenate([bat, pad_bat]).reshape(_NW * _NCHP, _KP)
    pool_src = jnp.concatenate(
        [jnp.arange(_N, dtype=jnp.int32),
         jnp.zeros((_PP - _N,), jnp.int32)]).reshape(_NW * _NCHP, _KP)

    x_pad = jnp.pad(x, ((0, _NP - _N), (0, 0)))

    zrows = jnp.zeros((_RPT, _H), jnp.float32)
    o128 = jnp.ones((_KE, _H), jnp.float32)

    deg_f, cnt_f = _histograms_sc(dst_p, bat_p, zrows, o128)
    deg3 = deg_f.reshape(_NC, _NP, _H)
    cnt3 = cnt_f.reshape(_NC, _CROWS, _H)[:, :_G]

    dinv_b, t0 = _prep_tc(deg3, x_pad, conv_W0)

    def b8(v):
        return jnp.tile(v[None, :], (8, 1))

    p0 = _segment_rows_sc(t0, src_p, dst_p, zrows, _NCHE, _KE).reshape(_NC, _NP, _H)
    t1 = _combine_mm(p0, t0, dinv_b, b8(conv_b0), conv_W1)
    p1 = _segment_rows_sc(t1, src_p, dst_p, zrows, _NCHE, _KE).reshape(_NC, _NP, _H)
    t2 = _combine_mm(p1, t1, dinv_b, b8(conv_b1), conv_W2)
    p2 = _segment_rows_sc(t2, src_p, dst_p, zrows, _NCHE, _KE).reshape(_NC, _NP, _H)
    h3 = _combine_id(p2, t2, dinv_b, b8(conv_b2))

    pp3 = _segment_rows_sc(h3, pool_src, bat_p, zrows, _NCHP, _KP)
    pp3 = pp3.reshape(_NC, _NP, _H)[:, :_G]

    return _final_tc(pp3, cnt3, mlp_W0, b8(mlp_b0), b8(bn_gamma),
                     b8(bn_beta), b8(bn_mean), b8(bn_var), mlp_W1, b8(mlp_b1))


# speedup vs baseline: 17.6775x; 2.8825x over previous
import functools

import jax
import jax.numpy as jnp
from jax import lax
from jax.experimental import pallas as pl
from jax.experimental.pallas import tpu as pltpu
from jax.experimental.pallas import tpu_sc as plsc

_N = 10000
_E = 320000
_D = 128
_H = 128
_HID = 64
_C = 10
_G = 256

_NP = 10240
_NC = 2
_NS = 16
_NW = _NC * _NS
_RPT = _NP // _NS

_EP = 327680
_KE = 128
_NCHE = (_EP // _NW) // _KE

_KP = 64
_NCHP = 8
_PP = _NW * _NCHP * _KP

_CROWS = 512

_RB = 256


def _segment_rows_sc(t, src2, dst2, zrows, nch, k):
    mesh = plsc.VectorSubcoreMesh(core_axis_name="c", subcore_axis_name="s")
    ib = min(16, nch)
    nblk = nch // ib

    @functools.partial(
        pl.kernel,
        out_type=jax.ShapeDtypeStruct((_NC * _NP, _H), jnp.float32),
        mesh=mesh,
        scratch_types=[
            pltpu.VMEM((ib, k), jnp.int32),
            pltpu.VMEM((ib, k), jnp.int32),
            pltpu.VMEM((k, _H), jnp.float32),
            pltpu.VMEM((k, _H), jnp.float32),
            pltpu.VMEM_SHARED((_NP, _H), jnp.float32),
            pltpu.SemaphoreType.DMA,
            pltpu.SemaphoreType.DMA,
        ],
    )
    def body(t_hbm, src_hbm, dst_hbm, z_hbm, out_hbm, src_v, dst_v, buf_a,
             buf_b, acc, sem_a, sem_b):
        cid = lax.axis_index("c")
        sid = lax.axis_index("s")
        wid = cid * _NS + sid

        pltpu.sync_copy(z_hbm, acc.at[pl.ds(sid * _RPT, _RPT)])

        plsc.subcore_barrier()

        @pl.loop(0, nblk)
        def _(b):
            base = wid * nch + b * ib
            pltpu.async_copy(src_hbm.at[pl.ds(base, ib)], src_v, sem_a).wait()
            pltpu.async_copy(dst_hbm.at[pl.ds(base, ib)], dst_v, sem_a).wait()
            pltpu.async_copy(t_hbm.at[src_v.at[0]], buf_a, sem_a)

            @pl.loop(0, ib // 2)
            def _(jh):
                j = jh * 2
                pltpu.make_async_copy(t_hbm.at[src_v.at[0]], buf_a, sem_a).wait()
                pltpu.async_copy(t_hbm.at[src_v.at[j + 1]], buf_b, sem_b)
                pltpu.sync_copy(buf_a, acc.at[dst_v.at[j]], add=True)
                pltpu.make_async_copy(t_hbm.at[src_v.at[0]], buf_b, sem_b).wait()

                @pl.when(j + 2 < ib)
                def _():
                    pltpu.async_copy(t_hbm.at[src_v.at[j + 2]], buf_a, sem_a)

                pltpu.sync_copy(buf_b, acc.at[dst_v.at[j + 1]], add=True)

        plsc.subcore_barrier()
        pltpu.sync_copy(acc.at[pl.ds(sid * _RPT, _RPT)],
                        out_hbm.at[pl.ds(cid * _NP + sid * _RPT, _RPT)])

    return body(t, src2, dst2, zrows)


def _histograms_sc(dst2, bat2, z16, o16):
    mesh = plsc.VectorSubcoreMesh(core_axis_name="c", subcore_axis_name="s")

    @functools.partial(
        pl.kernel,
        out_type=[jax.ShapeDtypeStruct((_NC * _NP, _H), jnp.float32),
                  jax.ShapeDtypeStruct((_NC * _CROWS, _H), jnp.float32)],
        mesh=mesh,
        scratch_types=[
            pltpu.VMEM((_NCHE, _KE), jnp.int32),
            pltpu.VMEM((_NCHP, _KP), jnp.int32),
            pltpu.VMEM((_KE, _H), jnp.float32),
            pltpu.VMEM_SHARED((_NP, _H), jnp.float32),
            pltpu.VMEM_SHARED((_CROWS, _H), jnp.float32),
            pltpu.SemaphoreType.DMA,
        ],
    )
    def body(dst_hbm, b_hbm, z_hbm, o_hbm, deg_hbm, cnt_hbm,
             dst_v, b_v, ones_v, dacc, cacc, sem):
        cid = lax.axis_index("c")
        sid = lax.axis_index("s")
        wid = cid * _NS + sid
        crpt = _CROWS // _NS

        pltpu.sync_copy(o_hbm, ones_v)
        pltpu.sync_copy(z_hbm, dacc.at[pl.ds(sid * _RPT, _RPT)])
        pltpu.sync_copy(z_hbm.at[pl.ds(0, crpt)],
                        cacc.at[pl.ds(sid * crpt, crpt)])

        plsc.subcore_barrier()

        pltpu.async_copy(dst_hbm.at[pl.ds(wid * _NCHE, _NCHE)], dst_v, sem).wait()
        pltpu.async_copy(b_hbm.at[pl.ds(wid * _NCHP, _NCHP)], b_v, sem).wait()

        @pl.loop(0, _NCHE)
        def _(j):
            pltpu.sync_copy(ones_v, dacc.at[dst_v.at[j]], add=True)

        @pl.loop(0, _NCHP)
        def _(j):
            pltpu.sync_copy(ones_v.at[pl.ds(0, _KP)], cacc.at[b_v.at[j]], add=True)

        plsc.subcore_barrier()
        pltpu.sync_copy(dacc.at[pl.ds(sid * _RPT, _RPT)],
                        deg_hbm.at[pl.ds(cid * _NP + sid * _RPT, _RPT)])
        pltpu.sync_copy(cacc.at[pl.ds(sid * crpt, crpt)],
                        cnt_hbm.at[pl.ds(cid * _CROWS + sid * crpt, crpt)])

    return body(dst2, bat2, z16, o16)


def _prep_body(deg_ref, x_ref, w_ref, dinv_ref, t0_ref):
    deg = deg_ref[0, :, 0:1] + deg_ref[1, :, 0:1] + 1.0
    db = jnp.broadcast_to(lax.rsqrt(deg), (_RB, _H))
    dinv_ref[...] = db
    t0_ref[...] = db * jnp.dot(x_ref[...], w_ref[...],
                               preferred_element_type=jnp.float32)


def _prep_tc(deg3, x_pad, w0):
    return pl.pallas_call(
        _prep_body,
        grid=(_NP // _RB,),
        in_specs=[pl.BlockSpec((_NC, _RB, _H), lambda i: (0, i, 0)),
                  pl.BlockSpec((_RB, _D), lambda i: (i, 0)),
                  pl.BlockSpec((_D, _H), lambda i: (0, 0))],
        out_specs=[pl.BlockSpec((_RB, _H), lambda i: (i, 0)),
                   pl.BlockSpec((_RB, _H), lambda i: (i, 0))],
        out_shape=[jax.ShapeDtypeStruct((_NP, _H), jnp.float32)] * 2,
    )(deg3, x_pad, w0)


def _combine_mm_body(p_ref, t_ref, dinv_ref, b_ref, w_ref, o_ref):
    s = p_ref[0] + p_ref[1] + t_ref[...]
    h = jnp.maximum(dinv_ref[...] * s + b_ref[0:1, :], 0.0)
    o_ref[...] = dinv_ref[...] * jnp.dot(h, w_ref[...],
                                         preferred_element_type=jnp.float32)


def _combine_mm(p3, t, dinv_b, b8, w):
    return pl.pallas_call(
        _combine_mm_body,
        grid=(_NP // _RB,),
        in_specs=[pl.BlockSpec((_NC, _RB, _H), lambda i: (0, i, 0)),
                  pl.BlockSpec((_RB, _H), lambda i: (i, 0)),
                  pl.BlockSpec((_RB, _H), lambda i: (i, 0)),
                  pl.BlockSpec((8, _H), lambda i: (0, 0)),
                  pl.BlockSpec((_H, _H), lambda i: (0, 0))],
        out_specs=pl.BlockSpec((_RB, _H), lambda i: (i, 0)),
        out_shape=jax.ShapeDtypeStruct((_NP, _H), jnp.float32),
    )(p3, t, dinv_b, b8, w)


def _combine_id_body(p_ref, t_ref, dinv_ref, b_ref, o_ref):
    s = p_ref[0] + p_ref[1] + t_ref[...]
    o_ref[...] = jnp.maximum(dinv_ref[...] * s + b_ref[0:1, :], 0.0)


def _combine_id(p3, t, dinv_b, b8):
    return pl.pallas_call(
        _combine_id_body,
        grid=(_NP // _RB,),
        in_specs=[pl.BlockSpec((_NC, _RB, _H), lambda i: (0, i, 0)),
                  pl.BlockSpec((_RB, _H), lambda i: (i, 0)),
                  pl.BlockSpec((_RB, _H), lambda i: (i, 0)),
                  pl.BlockSpec((8, _H), lambda i: (0, 0))],
        out_specs=pl.BlockSpec((_RB, _H), lambda i: (i, 0)),
        out_shape=jax.ShapeDtypeStruct((_NP, _H), jnp.float32),
    )(p3, t, dinv_b, b8)


def _final_body(p_ref, c_ref, w0_ref, b0_ref, gam_ref, bet_ref, mu_ref,
                var_ref, w1_ref, b1_ref, o_ref):
    pooled = p_ref[0] + p_ref[1]
    cnt = c_ref[0, :, 0:1] + c_ref[1, :, 0:1]
    mean = pooled / jnp.maximum(cnt, 1.0)
    z = jnp.dot(mean, w0_ref[...], preferred_element_type=jnp.float32) + b0_ref[0:1, :]
    z = (z - mu_ref[0:1, :]) * lax.rsqrt(var_ref[0:1, :] + 1e-5) * gam_ref[0:1, :] + bet_ref[0:1, :]
    z = jnp.maximum(z, 0.0)
    out = jnp.dot(z, w1_ref[...], preferred_element_type=jnp.float32) + b1_ref[0:1, :]
    m = jnp.max(out, axis=1, keepdims=True)
    lse = jnp.log(jnp.sum(jnp.exp(out - m), axis=1, keepdims=True)) + m
    o_ref[...] = out - lse


def _final_tc(pp3, cnt3, w0, b0, gam, bet, mu, var, w1, b1):
    return pl.pallas_call(
        _final_body,
        out_shape=jax.ShapeDtypeStruct((_G, _C), jnp.float32),
    )(pp3, cnt3, w0, b0, gam, bet, mu, var, w1, b1)


def kernel(x, edge_index, batch, conv_W0, conv_b0, conv_W1, conv_b1,
           conv_W2, conv_b2, mlp_W0, mlp_b0, bn_gamma, bn_beta, bn_mean,
           bn_var, mlp_W1, mlp_b1):
    src = edge_index[0].astype(jnp.int32)
    dst = edge_index[1].astype(jnp.int32)
    pad_e = _EP - _E
    pad_src = jnp.arange(pad_e, dtype=jnp.int32) % _N
    src_p = jnp.concatenate([src, pad_src]).reshape(_NW * _NCHE, _KE)
    pad_dst = _N + jnp.arange(pad_e, dtype=jnp.int32) % (_NP - _N)
    dst_p = jnp.concatenate([dst, pad_dst]).reshape(_NW * _NCHE, _KE)

    bat = batch.astype(jnp.int32)
    pad_bat = _G + jnp.arange(_PP - _N, dtype=jnp.int32) % (_CROWS - _G)
    bat_p = jnp.concatenate([bat, pad_bat]).reshape(_NW * _NCHP, _KP)
    pool_src = jnp.concatenate(
        [jnp.arange(_N, dtype=jnp.int32),
         jnp.arange(_PP - _N, dtype=jnp.int32) % _N]).reshape(_NW * _NCHP, _KP)

    x_pad = jnp.pad(x, ((0, _NP - _N), (0, 0)))

    zrows = jnp.zeros((_RPT, _H), jnp.float32)
    o128 = jnp.ones((_KE, _H), jnp.float32)

    deg_f, cnt_f = _histograms_sc(dst_p, bat_p, zrows, o128)
    deg3 = deg_f.reshape(_NC, _NP, _H)
    cnt3 = cnt_f.reshape(_NC, _CROWS, _H)[:, :_G]

    dinv_b, t0 = _prep_tc(deg3, x_pad, conv_W0)

    def b8(v):
        return jnp.tile(v[None, :], (8, 1))

    p0 = _segment_rows_sc(t0, src_p, dst_p, zrows, _NCHE, _KE).reshape(_NC, _NP, _H)
    t1 = _combine_mm(p0, t0, dinv_b, b8(conv_b0), conv_W1)
    p1 = _segment_rows_sc(t1, src_p, dst_p, zrows, _NCHE, _KE).reshape(_NC, _NP, _H)
    t2 = _combine_mm(p1, t1, dinv_b, b8(conv_b1), conv_W2)
    p2 = _segment_rows_sc(t2, src_p, dst_p, zrows, _NCHE, _KE).reshape(_NC, _NP, _H)
    h3 = _combine_id(p2, t2, dinv_b, b8(conv_b2))

    pp3 = _segment_rows_sc(h3, pool_src, bat_p, zrows, _NCHP, _KP)
    pp3 = pp3.reshape(_NC, _NP, _H)[:, :_G]

    return _final_tc(pp3, cnt3, mlp_W0, b8(mlp_b0), b8(bn_gamma),
                     b8(bn_beta), b8(bn_mean), b8(bn_var), mlp_W1, b8(mlp_b1))


# speedup vs baseline: 20.3284x; 1.1500x over previous
import functools

import jax
import jax.numpy as jnp
from jax import lax
from jax.experimental import pallas as pl
from jax.experimental.pallas import tpu as pltpu
from jax.experimental.pallas import tpu_sc as plsc

_N = 10000
_E = 320000
_D = 128
_H = 128
_HID = 64
_C = 10
_G = 256

_NP = 10240
_NC = 2
_NS = 16
_NW = _NC * _NS
_RPT = _NP // _NS

_EP = 327680
_KE = 64
_NCHE = (_EP // _NW) // _KE

_KP = 64
_NCHP = 8
_PP = _NW * _NCHP * _KP

_CROWS = 512

_RB = 256


def _segment_rows_sc(t, src2, dst2, zrows, nch, k):
    mesh = plsc.VectorSubcoreMesh(core_axis_name="c", subcore_axis_name="s")
    ib = min(32, nch)
    nblk = nch // ib
    nbuf = 4

    @functools.partial(
        pl.kernel,
        out_type=jax.ShapeDtypeStruct((_NC * _NP, _H), jnp.float32),
        mesh=mesh,
        scratch_types=[
            pltpu.VMEM((ib, k), jnp.int32),
            pltpu.VMEM((ib, k), jnp.int32),
            pltpu.VMEM((k, _H), jnp.float32),
            pltpu.VMEM((k, _H), jnp.float32),
            pltpu.VMEM((k, _H), jnp.float32),
            pltpu.VMEM((k, _H), jnp.float32),
            pltpu.VMEM_SHARED((_NP, _H), jnp.float32),
            pltpu.SemaphoreType.DMA,
            pltpu.SemaphoreType.DMA,
            pltpu.SemaphoreType.DMA,
            pltpu.SemaphoreType.DMA,
        ],
    )
    def body(t_hbm, src_hbm, dst_hbm, z_hbm, out_hbm, src_v, dst_v,
             b0, b1, b2, b3, acc, s0, s1, s2, s3):
        bufs = (b0, b1, b2, b3)
        sems = (s0, s1, s2, s3)
        cid = lax.axis_index("c")
        sid = lax.axis_index("s")
        wid = cid * _NS + sid

        pltpu.sync_copy(z_hbm, acc.at[pl.ds(sid * _RPT, _RPT)])

        plsc.subcore_barrier()

        @pl.loop(0, nblk)
        def _(blk):
            base = wid * nch + blk * ib
            pltpu.async_copy(src_hbm.at[pl.ds(base, ib)], src_v, s0).wait()
            pltpu.async_copy(dst_hbm.at[pl.ds(base, ib)], dst_v, s0).wait()
            for r in range(nbuf):
                pltpu.async_copy(t_hbm.at[src_v.at[r]], bufs[r], sems[r])

            @pl.loop(0, ib // nbuf)
            def _(g):
                for r in range(nbuf):
                    j = g * nbuf + r
                    pltpu.make_async_copy(
                        t_hbm.at[src_v.at[0]], bufs[r], sems[r]).wait()
                    pltpu.sync_copy(bufs[r], acc.at[dst_v.at[j]], add=True)

                    @pl.when(j + nbuf < ib)
                    def _():
                        pltpu.async_copy(
                            t_hbm.at[src_v.at[j + nbuf]], bufs[r], sems[r])

        plsc.subcore_barrier()
        pltpu.sync_copy(acc.at[pl.ds(sid * _RPT, _RPT)],
                        out_hbm.at[pl.ds(cid * _NP + sid * _RPT, _RPT)])

    return body(t, src2, dst2, zrows)


def _histograms_sc(dst2, bat2, z16, o16):
    mesh = plsc.VectorSubcoreMesh(core_axis_name="c", subcore_axis_name="s")

    @functools.partial(
        pl.kernel,
        out_type=[jax.ShapeDtypeStruct((_NC * _NP, _H), jnp.float32),
                  jax.ShapeDtypeStruct((_NC * _CROWS, _H), jnp.float32)],
        mesh=mesh,
        scratch_types=[
            pltpu.VMEM((_NCHE, _KE), jnp.int32),
            pltpu.VMEM((_NCHP, _KP), jnp.int32),
            pltpu.VMEM((_KE, _H), jnp.float32),
            pltpu.VMEM_SHARED((_NP, _H), jnp.float32),
            pltpu.VMEM_SHARED((_CROWS, _H), jnp.float32),
            pltpu.SemaphoreType.DMA,
        ],
    )
    def body(dst_hbm, b_hbm, z_hbm, o_hbm, deg_hbm, cnt_hbm,
             dst_v, b_v, ones_v, dacc, cacc, sem):
        cid = lax.axis_index("c")
        sid = lax.axis_index("s")
        wid = cid * _NS + sid
        crpt = _CROWS // _NS

        pltpu.sync_copy(o_hbm, ones_v)
        pltpu.sync_copy(z_hbm, dacc.at[pl.ds(sid * _RPT, _RPT)])
        pltpu.sync_copy(z_hbm.at[pl.ds(0, crpt)],
                        cacc.at[pl.ds(sid * crpt, crpt)])

        plsc.subcore_barrier()

        pltpu.async_copy(dst_hbm.at[pl.ds(wid * _NCHE, _NCHE)], dst_v, sem).wait()
        pltpu.async_copy(b_hbm.at[pl.ds(wid * _NCHP, _NCHP)], b_v, sem).wait()

        @pl.loop(0, _NCHE)
        def _(j):
            pltpu.sync_copy(ones_v, dacc.at[dst_v.at[j]], add=True)

        @pl.loop(0, _NCHP)
        def _(j):
            pltpu.sync_copy(ones_v.at[pl.ds(0, _KP)], cacc.at[b_v.at[j]], add=True)

        plsc.subcore_barrier()
        pltpu.sync_copy(dacc.at[pl.ds(sid * _RPT, _RPT)],
                        deg_hbm.at[pl.ds(cid * _NP + sid * _RPT, _RPT)])
        pltpu.sync_copy(cacc.at[pl.ds(sid * crpt, crpt)],
                        cnt_hbm.at[pl.ds(cid * _CROWS + sid * crpt, crpt)])

    return body(dst2, bat2, z16, o16)


def _prep_body(deg_ref, x_ref, w_ref, dinv_ref, t0_ref):
    deg = deg_ref[0, :, 0:1] + deg_ref[1, :, 0:1] + 1.0
    db = jnp.broadcast_to(lax.rsqrt(deg), (_RB, _H))
    dinv_ref[...] = db
    t0_ref[...] = db * jnp.dot(x_ref[...], w_ref[...],
                               preferred_element_type=jnp.float32)


def _prep_tc(deg3, x_pad, w0):
    return pl.pallas_call(
        _prep_body,
        grid=(_NP // _RB,),
        in_specs=[pl.BlockSpec((_NC, _RB, _H), lambda i: (0, i, 0)),
                  pl.BlockSpec((_RB, _D), lambda i: (i, 0)),
                  pl.BlockSpec((_D, _H), lambda i: (0, 0))],
        out_specs=[pl.BlockSpec((_RB, _H), lambda i: (i, 0)),
                   pl.BlockSpec((_RB, _H), lambda i: (i, 0))],
        out_shape=[jax.ShapeDtypeStruct((_NP, _H), jnp.float32)] * 2,
    )(deg3, x_pad, w0)


def _combine_mm_body(p_ref, t_ref, dinv_ref, b_ref, w_ref, o_ref):
    s = p_ref[0] + p_ref[1] + t_ref[...]
    h = jnp.maximum(dinv_ref[...] * s + b_ref[0:1, :], 0.0)
    o_ref[...] = dinv_ref[...] * jnp.dot(h, w_ref[...],
                                         preferred_element_type=jnp.float32)


def _combine_mm(p3, t, dinv_b, b8, w):
    return pl.pallas_call(
        _combine_mm_body,
        grid=(_NP // _RB,),
        in_specs=[pl.BlockSpec((_NC, _RB, _H), lambda i: (0, i, 0)),
                  pl.BlockSpec((_RB, _H), lambda i: (i, 0)),
                  pl.BlockSpec((_RB, _H), lambda i: (i, 0)),
                  pl.BlockSpec((8, _H), lambda i: (0, 0)),
                  pl.BlockSpec((_H, _H), lambda i: (0, 0))],
        out_specs=pl.BlockSpec((_RB, _H), lambda i: (i, 0)),
        out_shape=jax.ShapeDtypeStruct((_NP, _H), jnp.float32),
    )(p3, t, dinv_b, b8, w)


def _combine_id_body(p_ref, t_ref, dinv_ref, b_ref, o_ref):
    s = p_ref[0] + p_ref[1] + t_ref[...]
    o_ref[...] = jnp.maximum(dinv_ref[...] * s + b_ref[0:1, :], 0.0)


def _combine_id(p3, t, dinv_b, b8):
    return pl.pallas_call(
        _combine_id_body,
        grid=(_NP // _RB,),
        in_specs=[pl.BlockSpec((_NC, _RB, _H), lambda i: (0, i, 0)),
                  pl.BlockSpec((_RB, _H), lambda i: (i, 0)),
                  pl.BlockSpec((_RB, _H), lambda i: (i, 0)),
                  pl.BlockSpec((8, _H), lambda i: (0, 0))],
        out_specs=pl.BlockSpec((_RB, _H), lambda i: (i, 0)),
        out_shape=jax.ShapeDtypeStruct((_NP, _H), jnp.float32),
    )(p3, t, dinv_b, b8)


def _final_body(p_ref, c_ref, w0_ref, b0_ref, gam_ref, bet_ref, mu_ref,
                var_ref, w1_ref, b1_ref, o_ref):
    pooled = p_ref[0] + p_ref[1]
    cnt = c_ref[0, :, 0:1] + c_ref[1, :, 0:1]
    mean = pooled / jnp.maximum(cnt, 1.0)
    z = jnp.dot(mean, w0_ref[...], preferred_element_type=jnp.float32) + b0_ref[0:1, :]
    z = (z - mu_ref[0:1, :]) * lax.rsqrt(var_ref[0:1, :] + 1e-5) * gam_ref[0:1, :] + bet_ref[0:1, :]
    z = jnp.maximum(z, 0.0)
    out = jnp.dot(z, w1_ref[...], preferred_element_type=jnp.float32) + b1_ref[0:1, :]
    m = jnp.max(out, axis=1, keepdims=True)
    lse = jnp.log(jnp.sum(jnp.exp(out - m), axis=1, keepdims=True)) + m
    o_ref[...] = out - lse


def _final_tc(pp3, cnt3, w0, b0, gam, bet, mu, var, w1, b1):
    return pl.pallas_call(
        _final_body,
        out_shape=jax.ShapeDtypeStruct((_G, _C), jnp.float32),
    )(pp3, cnt3, w0, b0, gam, bet, mu, var, w1, b1)


def kernel(x, edge_index, batch, conv_W0, conv_b0, conv_W1, conv_b1,
           conv_W2, conv_b2, mlp_W0, mlp_b0, bn_gamma, bn_beta, bn_mean,
           bn_var, mlp_W1, mlp_b1):
    src = edge_index[0].astype(jnp.int32)
    dst = edge_index[1].astype(jnp.int32)
    pad_e = _EP - _E
    pad_src = jnp.arange(pad_e, dtype=jnp.int32) % _N
    src_p = jnp.concatenate([src, pad_src]).reshape(_NW * _NCHE, _KE)
    pad_dst = _N + jnp.arange(pad_e, dtype=jnp.int32) % (_NP - _N)
    dst_p = jnp.concatenate([dst, pad_dst]).reshape(_NW * _NCHE, _KE)

    bat = batch.astype(jnp.int32)
    pad_bat = _G + jnp.arange(_PP - _N, dtype=jnp.int32) % (_CROWS - _G)
    bat_p = jnp.concatenate([bat, pad_bat]).reshape(_NW * _NCHP, _KP)
    pool_src = jnp.concatenate(
        [jnp.arange(_N, dtype=jnp.int32),
         jnp.arange(_PP - _N, dtype=jnp.int32) % _N]).reshape(_NW * _NCHP, _KP)

    x_pad = jnp.pad(x, ((0, _NP - _N), (0, 0)))

    zrows = jnp.zeros((_RPT, _H), jnp.float32)
    o128 = jnp.ones((_KE, _H), jnp.float32)

    deg_f, cnt_f = _histograms_sc(dst_p, bat_p, zrows, o128)
    deg3 = deg_f.reshape(_NC, _NP, _H)
    cnt3 = cnt_f.reshape(_NC, _CROWS, _H)[:, :_G]

    dinv_b, t0 = _prep_tc(deg3, x_pad, conv_W0)

    def b8(v):
        return jnp.tile(v[None, :], (8, 1))

    p0 = _segment_rows_sc(t0, src_p, dst_p, zrows, _NCHE, _KE).reshape(_NC, _NP, _H)
    t1 = _combine_mm(p0, t0, dinv_b, b8(conv_b0), conv_W1)
    p1 = _segment_rows_sc(t1, src_p, dst_p, zrows, _NCHE, _KE).reshape(_NC, _NP, _H)
    t2 = _combine_mm(p1, t1, dinv_b, b8(conv_b1), conv_W2)
    p2 = _segment_rows_sc(t2, src_p, dst_p, zrows, _NCHE, _KE).reshape(_NC, _NP, _H)
    h3 = _combine_id(p2, t2, dinv_b, b8(conv_b2))

    pp3 = _segment_rows_sc(h3, pool_src, bat_p, zrows, _NCHP, _KP)
    pp3 = pp3.reshape(_NC, _NP, _H)[:, :_G]

    return _final_tc(pp3, cnt3, mlp_W0, b8(mlp_b0), b8(bn_gamma),
                     b8(bn_beta), b8(bn_mean), b8(bn_var), mlp_W1, b8(mlp_b1))
